# baseline (speedup 1.0000x reference)
import functools

import jax
import jax.numpy as jnp
from jax import lax
from jax.experimental import pallas as pl
from jax.experimental.pallas import tpu as pltpu
from jax.experimental.pallas import tpu_sc as plsc

_NC = 2
_NS = 16
_LANES = 16
_CHUNK = 128
_FAST_FRAC = 0.5


def _split_chunks(total_chunks):
    per_tile = total_chunks // _NS
    ch0 = int(round(_FAST_FRAC * per_tile / 8.0)) * 8
    ch0 = max(8, min(per_tile - 8, ch0))
    return ch0, per_tile - ch0


def _stages(ch, cap):
    out = []
    off = 0
    while off < ch:
        n = min(cap, ch - off)
        out.append((off, n))
        off += n
    return out


@functools.partial(jax.jit, static_argnames=("n_pad", "n_out", "ch0", "ch1",
                                             "with_cnt"))
def _sc_aggregate(src2d, dst2d, x, *, n_pad, n_out, ch0, ch1, with_cnt):
    n_nodes, d = x.shape
    assert ch0 % 8 == 0 and ch1 % 8 == 0
    idx_cap = 64
    rpt = n_pad // _NS
    assert rpt % 8 == 0
    nz = (rpt + _CHUNK - 1) // _CHUNK

    mesh = plsc.VectorSubcoreMesh(core_axis_name="c", subcore_axis_name="s")

    out_type = [jax.ShapeDtypeStruct((_NC, n_out, d), jnp.float32)]
    if with_cnt:
        out_type.append(jax.ShapeDtypeStruct((_NC, n_out), jnp.float32))

    @functools.partial(
        pl.kernel,
        out_type=tuple(out_type),
        mesh=mesh,
        scratch_types=[
            pltpu.VMEM((idx_cap, _CHUNK), jnp.int32),
            pltpu.VMEM((idx_cap, _CHUNK), jnp.int32),
            pltpu.VMEM((_CHUNK, d), jnp.float32),
            pltpu.VMEM((_CHUNK, d), jnp.float32),
            pltpu.VMEM((_CHUNK,), jnp.float32),
            pltpu.VMEM_SHARED((n_pad, d), jnp.float32),
            pltpu.VMEM_SHARED((n_pad,), jnp.float32),
            pltpu.SemaphoreType.DMA,
            pltpu.SemaphoreType.DMA,
        ],
    )
    def agg_kernel(src_hbm, dst_hbm, x_hbm, agg_hbm, *rest):
        if with_cnt:
            cnt_hbm = rest[0]
            rest = rest[1:]
        sidx, didx, rows0, rows1, ones, acc, cacc, gsem, ssem = rest
        c = lax.axis_index("c")
        s = lax.axis_index("s")

        scope0 = jax.named_scope("init_zero")
        scope0.__enter__()

        @pl.loop(0, _CHUNK)
        def _(r):
            @pl.loop(0, d, step=_LANES)
            def _(col):
                rows0[r, pl.ds(col, _LANES)] = jnp.zeros((_LANES,),
                                                         jnp.float32)

        if with_cnt:
            @pl.loop(0, _CHUNK, step=_LANES)
            def _(i):
                ones[pl.ds(i, _LANES)] = jnp.ones((_LANES,), jnp.float32)

        @pl.loop(0, nz)
        def _(j):
            base = jnp.minimum(s * rpt + j * _CHUNK, n_pad - _CHUNK)
            pltpu.sync_copy(rows0, acc.at[pl.ds(base, _CHUNK)])
            if with_cnt:
                pltpu.sync_copy(rows0.at[0], cacc.at[pl.ds(base, _CHUNK)])

        plsc.subcore_barrier()
        scope0.__exit__(None, None, None)

        def fire_gather(t, buf):
            pltpu.async_copy(x_hbm.at[sidx.at[t]], buf, gsem)

        def drain_gather(t, buf):
            pltpu.make_async_copy(x_hbm.at[sidx.at[t]], buf, gsem).wait()

        def scatter_group(t, buf):
            descs = [pltpu.async_copy(buf, acc.at[didx.at[t]], ssem,
                                      add=True)]
            if with_cnt:
                descs.append(pltpu.async_copy(ones, cacc.at[didx.at[t]],
                                              ssem, add=True))
            for dsc in descs:
                dsc.wait()

        def run_stage(chunk_base, n_chunks):
            pltpu.sync_copy(src_hbm.at[pl.ds(chunk_base, n_chunks)],
                            sidx.at[pl.ds(0, n_chunks)])
            pltpu.sync_copy(dst_hbm.at[pl.ds(chunk_base, n_chunks)],
                            didx.at[pl.ds(0, n_chunks)])
            fire_gather(0, rows0)

            @pl.loop(0, n_chunks, step=2)
            def _(p):
                drain_gather(p, rows0)
                fire_gather(p + 1, rows1)
                scatter_group(p, rows0)
                drain_gather(p + 1, rows1)

                @pl.when(p + 2 < n_chunks)
                def _():
                    fire_gather(p + 2, rows0)

                scatter_group(p + 1, rows1)

        with jax.named_scope("edge_loop"):
            @pl.when(c == 0)
            def _():
                for off, cnt_c in _stages(ch0, idx_cap):
                    run_stage(s * ch0 + off, cnt_c)

            @pl.when(c == 1)
            def _():
                for off, cnt_c in _stages(ch1, idx_cap):
                    run_stage(_NS * ch0 + s * ch1 + off, cnt_c)

        plsc.subcore_barrier()

        with jax.named_scope("writeout"):
            pltpu.sync_copy(acc.at[pl.ds(s * rpt, rpt)],
                            agg_hbm.at[c, pl.ds(s * rpt, rpt)])
            if with_cnt:
                @pl.when(s == 0)
                def _():
                    pltpu.sync_copy(cacc, cnt_hbm.at[c, pl.ds(0, n_pad)])

    return agg_kernel(src2d, dst2d, x)


def _dense_layer1(agg, cnt, x, w_l, b_l, w_r, b_r, gamma, beta):
    n, d = x.shape

    def body(agg_ref, cnt_ref, x_ref, wl_ref, bl_ref, wr_ref, br_ref,
             g_ref, be_ref, h_ref):
        cntv = jnp.maximum(cnt_ref[0, :n] + cnt_ref[1, :n], 1.0)
        mean = (agg_ref[0, :n, :] + agg_ref[1, :n, :]) / cntv[:, None]
        h = (jnp.dot(mean, wl_ref[...], preferred_element_type=jnp.float32,
                     precision=lax.Precision.HIGHEST)
             + jnp.dot(x_ref[...], wr_ref[...],
                       preferred_element_type=jnp.float32,
                       precision=lax.Precision.HIGHEST)
             + bl_ref[...] + br_ref[...])
        mu = jnp.mean(h, axis=0, keepdims=True)
        var = jnp.mean((h - mu) * (h - mu), axis=0, keepdims=True)
        hn = (h - mu) * lax.rsqrt(var + 1e-5) * g_ref[...] + be_ref[...]
        h_ref[...] = jax.nn.gelu(hn)

    return pl.pallas_call(
        body,
        out_shape=jax.ShapeDtypeStruct((n, d), jnp.float32),
    )(agg, cnt, x, w_l, b_l, w_r, b_r, gamma, beta)


def _dense_layer2(agg, cnt, h, w_l, b_l, w_r, b_r):
    n, d = h.shape

    def body(agg_ref, cnt_ref, h_ref, wl_ref, bl_ref, wr_ref, br_ref,
             o_ref):
        cntv = jnp.maximum(cnt_ref[0, :n] + cnt_ref[1, :n], 1.0)
        mean = (agg_ref[0, :n, :] + agg_ref[1, :n, :]) / cntv[:, None]
        o_ref[...] = (jnp.dot(mean, wl_ref[...],
                              preferred_element_type=jnp.float32,
                              precision=lax.Precision.HIGHEST)
                      + jnp.dot(h_ref[...], wr_ref[...],
                                preferred_element_type=jnp.float32,
                                precision=lax.Precision.HIGHEST)
                      + bl_ref[...] + br_ref[...])

    return pl.pallas_call(
        body,
        out_shape=jax.ShapeDtypeStruct((n, d), jnp.float32),
    )(agg, cnt, h, w_l, b_l, w_r, b_r)


def kernel(x, edge_index, W_l1, b_l1, W_r1, b_r1, gamma1, beta1,
           W_l2, b_l2, W_r2, b_r2):
    n, d = x.shape
    e = edge_index.shape[1]

    n_pad = ((n + 16 + _CHUNK - 1) // _CHUNK) * _CHUNK
    n_out = ((n_pad + _NS * _CHUNK - 1) // (_NS * _CHUNK)) * (_NS * _CHUNK)
    nw = _NC * _NS
    unit = nw * _CHUNK * 8
    e_pad = ((e + unit - 1) // unit) * unit
    ch0, ch1 = _split_chunks(e_pad // _CHUNK)

    src = edge_index[0]
    dst = edge_index[1]
    if e_pad != e:
        pad = e_pad - e
        src = jnp.concatenate([src, jnp.zeros((pad,), jnp.int32)])
        trash = n + jnp.arange(pad, dtype=jnp.int32) % (n_pad - n)
        dst = jnp.concatenate([dst, trash])
    src2d = src.reshape(e_pad // _CHUNK, _CHUNK)
    dst2d = dst.reshape(e_pad // _CHUNK, _CHUNK)

    agg1, cnt = _sc_aggregate(src2d, dst2d, x, n_pad=n_pad, n_out=n_out,
                              ch0=ch0, ch1=ch1, with_cnt=True)
    h = _dense_layer1(agg1, cnt, x, W_l1, b_l1, W_r1, b_r1, gamma1, beta1)
    (agg2,) = _sc_aggregate(src2d, dst2d, h, n_pad=n_pad, n_out=n_out,
                            ch0=ch0, ch1=ch1, with_cnt=False)
    return _dense_layer2(agg2, cnt, h, W_l2, b_l2, W_r2, b_r2)


# speedup vs baseline: 3.0371x; 3.0371x over previous
import functools

import jax
import jax.numpy as jnp
from jax import lax
from jax.experimental import pallas as pl
from jax.experimental.pallas import tpu as pltpu
from jax.experimental.pallas import tpu_sc as plsc

_NC = 2
_NS = 16
_LANES = 16
_CHUNK = 128
_FAST_FRAC = 0.5


def _split_chunks(total_chunks):
    per_tile = total_chunks // _NS
    ch0 = int(round(_FAST_FRAC * per_tile / 8.0)) * 8
    ch0 = max(8, min(per_tile - 8, ch0))
    return ch0, per_tile - ch0


def _stages(ch, cap):
    out = []
    off = 0
    while off < ch:
        n = min(cap, ch - off)
        out.append((off, n))
        off += n
    return out


@functools.partial(jax.jit, static_argnames=("n_pad", "n_out", "ch0", "ch1",
                                             "real_chunks", "with_cnt"))
def _sc_aggregate(src2d, dst2d, x, *, n_pad, n_out, ch0, ch1, real_chunks,
                  with_cnt):
    n_nodes, d = x.shape
    assert ch0 % 8 == 0 and ch1 % 8 == 0
    idx_cap = 64
    rpt = n_pad // _NS
    assert rpt % 8 == 0
    nz = (rpt + _CHUNK - 1) // _CHUNK

    mesh = plsc.VectorSubcoreMesh(core_axis_name="c", subcore_axis_name="s")

    out_type = [jax.ShapeDtypeStruct((_NC, n_out, d), jnp.float32)]
    if with_cnt:
        out_type.append(jax.ShapeDtypeStruct((_NC, n_out), jnp.float32))

    @functools.partial(
        pl.kernel,
        out_type=tuple(out_type),
        mesh=mesh,
        scratch_types=[
            pltpu.VMEM((idx_cap, _CHUNK), jnp.int32),
            pltpu.VMEM((idx_cap, _CHUNK), jnp.int32),
            pltpu.VMEM((_CHUNK, d), jnp.float32),
            pltpu.VMEM((_CHUNK, d), jnp.float32),
            pltpu.VMEM((_CHUNK,), jnp.float32),
            pltpu.VMEM_SHARED((n_pad, d), jnp.float32),
            pltpu.VMEM_SHARED((n_pad,), jnp.float32),
            pltpu.SemaphoreType.DMA,
            pltpu.SemaphoreType.DMA,
        ],
    )
    def agg_kernel(src_hbm, dst_hbm, x_hbm, agg_hbm, *rest):
        if with_cnt:
            cnt_hbm = rest[0]
            rest = rest[1:]
        sidx, didx, rows0, rows1, ones, acc, cacc, gsem, ssem = rest
        c = lax.axis_index("c")
        s = lax.axis_index("s")

        scope0 = jax.named_scope("init_zero")
        scope0.__enter__()

        @pl.loop(0, _CHUNK)
        def _(r):
            @pl.loop(0, d, step=_LANES)
            def _(col):
                rows0[r, pl.ds(col, _LANES)] = jnp.zeros((_LANES,),
                                                         jnp.float32)

        if with_cnt:
            @pl.loop(0, _CHUNK, step=_LANES)
            def _(i):
                ones[pl.ds(i, _LANES)] = jnp.ones((_LANES,), jnp.float32)

        @pl.loop(0, nz)
        def _(j):
            base = jnp.minimum(s * rpt + j * _CHUNK, n_pad - _CHUNK)
            pltpu.sync_copy(rows0, acc.at[pl.ds(base, _CHUNK)])
            if with_cnt:
                pltpu.sync_copy(rows0.at[0], cacc.at[pl.ds(base, _CHUNK)])

        plsc.subcore_barrier()
        scope0.__exit__(None, None, None)

        def fire_gather(t, buf):
            pltpu.async_copy(x_hbm.at[sidx.at[t]], buf, gsem)

        def drain_gather(t, buf):
            pltpu.make_async_copy(x_hbm.at[sidx.at[t]], buf, gsem).wait()

        def scatter_group(t, buf):
            descs = [pltpu.async_copy(buf, acc.at[didx.at[t]], ssem,
                                      add=True)]
            if with_cnt:
                descs.append(pltpu.async_copy(ones, cacc.at[didx.at[t]],
                                              ssem, add=True))
            for dsc in descs:
                dsc.wait()

        def run_stage(chunk_base, n_chunks, limit):
            pltpu.sync_copy(src_hbm.at[pl.ds(chunk_base, n_chunks)],
                            sidx.at[pl.ds(0, n_chunks)])
            pltpu.sync_copy(dst_hbm.at[pl.ds(chunk_base, n_chunks)],
                            didx.at[pl.ds(0, n_chunks)])

            @pl.when(limit > 0)
            def _():
                fire_gather(0, rows0)

            @pl.loop(0, n_chunks, step=2)
            def _(p):
                @pl.when(p < limit)
                def _():
                    drain_gather(p, rows0)

                @pl.when(p + 1 < limit)
                def _():
                    fire_gather(p + 1, rows1)

                @pl.when(p < limit)
                def _():
                    scatter_group(p, rows0)

                @pl.when(p + 1 < limit)
                def _():
                    drain_gather(p + 1, rows1)

                @pl.when(jnp.logical_and(p + 2 < n_chunks, p + 2 < limit))
                def _():
                    fire_gather(p + 2, rows0)

                @pl.when(p + 1 < limit)
                def _():
                    scatter_group(p + 1, rows1)

        with jax.named_scope("edge_loop"):
            @pl.when(c == 0)
            def _():
                for off, cnt_c in _stages(ch0, idx_cap):
                    base = s * ch0 + off
                    run_stage(base, cnt_c, real_chunks - base)

            @pl.when(c == 1)
            def _():
                for off, cnt_c in _stages(ch1, idx_cap):
                    base = _NS * ch0 + s * ch1 + off
                    run_stage(base, cnt_c, real_chunks - base)

        plsc.subcore_barrier()

        with jax.named_scope("writeout"):
            pltpu.sync_copy(acc.at[pl.ds(s * rpt, rpt)],
                            agg_hbm.at[c, pl.ds(s * rpt, rpt)])
            if with_cnt:
                @pl.when(s == 0)
                def _():
                    pltpu.sync_copy(cacc, cnt_hbm.at[c, pl.ds(0, n_pad)])

    return agg_kernel(src2d, dst2d, x)


def _dense_layer1(agg, cnt, x, w_l, b_l, w_r, b_r, gamma, beta):
    n, d = x.shape

    def body(agg_ref, cnt_ref, x_ref, wl_ref, bl_ref, wr_ref, br_ref,
             g_ref, be_ref, h_ref):
        cntv = jnp.maximum(cnt_ref[0, :n] + cnt_ref[1, :n], 1.0)
        mean = (agg_ref[0, :n, :] + agg_ref[1, :n, :]) / cntv[:, None]
        h = (jnp.dot(mean, wl_ref[...], preferred_element_type=jnp.float32,
                     precision=lax.Precision.HIGHEST)
             + jnp.dot(x_ref[...], wr_ref[...],
                       preferred_element_type=jnp.float32,
                       precision=lax.Precision.HIGHEST)
             + bl_ref[...] + br_ref[...])
        mu = jnp.mean(h, axis=0, keepdims=True)
        var = jnp.mean((h - mu) * (h - mu), axis=0, keepdims=True)
        hn = (h - mu) * lax.rsqrt(var + 1e-5) * g_ref[...] + be_ref[...]
        h_ref[...] = jax.nn.gelu(hn)

    return pl.pallas_call(
        body,
        out_shape=jax.ShapeDtypeStruct((n, d), jnp.float32),
    )(agg, cnt, x, w_l, b_l, w_r, b_r, gamma, beta)


def _dense_layer2(agg, cnt, h, w_l, b_l, w_r, b_r):
    n, d = h.shape

    def body(agg_ref, cnt_ref, h_ref, wl_ref, bl_ref, wr_ref, br_ref,
             o_ref):
        cntv = jnp.maximum(cnt_ref[0, :n] + cnt_ref[1, :n], 1.0)
        mean = (agg_ref[0, :n, :] + agg_ref[1, :n, :]) / cntv[:, None]
        o_ref[...] = (jnp.dot(mean, wl_ref[...],
                              preferred_element_type=jnp.float32,
                              precision=lax.Precision.HIGHEST)
                      + jnp.dot(h_ref[...], wr_ref[...],
                                preferred_element_type=jnp.float32,
                                precision=lax.Precision.HIGHEST)
                      + bl_ref[...] + br_ref[...])

    return pl.pallas_call(
        body,
        out_shape=jax.ShapeDtypeStruct((n, d), jnp.float32),
    )(agg, cnt, h, w_l, b_l, w_r, b_r)


def kernel(x, edge_index, W_l1, b_l1, W_r1, b_r1, gamma1, beta1,
           W_l2, b_l2, W_r2, b_r2):
    n, d = x.shape
    e = edge_index.shape[1]

    n_pad = ((n + 16 + _CHUNK - 1) // _CHUNK) * _CHUNK
    n_out = ((n_pad + _NS * _CHUNK - 1) // (_NS * _CHUNK)) * (_NS * _CHUNK)
    nw = _NC * _NS
    unit = nw * _CHUNK * 8
    e_pad = ((e + unit - 1) // unit) * unit
    ch0, ch1 = _split_chunks(e_pad // _CHUNK)

    src = edge_index[0]
    dst = edge_index[1]
    if e_pad != e:
        pad = e_pad - e
        src = jnp.concatenate([src, jnp.zeros((pad,), jnp.int32)])
        trash = n + jnp.arange(pad, dtype=jnp.int32) % (n_pad - n)
        dst = jnp.concatenate([dst, trash])
    src2d = src.reshape(e_pad // _CHUNK, _CHUNK)
    dst2d = dst.reshape(e_pad // _CHUNK, _CHUNK)

    real_chunks = (e + _CHUNK - 1) // _CHUNK
    agg1, cnt = _sc_aggregate(src2d, dst2d, x, n_pad=n_pad, n_out=n_out,
                              ch0=ch0, ch1=ch1, real_chunks=real_chunks,
                              with_cnt=True)
    h = _dense_layer1(agg1, cnt, x, W_l1, b_l1, W_r1, b_r1, gamma1, beta1)
    (agg2,) = _sc_aggregate(src2d, dst2d, h, n_pad=n_pad, n_out=n_out,
                            ch0=ch0, ch1=ch1, real_chunks=real_chunks,
                            with_cnt=False)
    return _dense_layer2(agg2, cnt, h, W_l2, b_l2, W_r2, b_r2)


# speedup vs baseline: 3.1935x; 1.0515x over previous
import functools

import jax
import jax.numpy as jnp
from jax import lax
from jax.experimental import pallas as pl
from jax.experimental.pallas import tpu as pltpu
from jax.experimental.pallas import tpu_sc as plsc

_NC = 2
_NS = 16
_LANES = 16
_CHUNK = 128
_FAST_FRAC = 0.5


def _split_chunks(total_chunks):
    per_tile = total_chunks // _NS
    ch0 = int(round(_FAST_FRAC * per_tile / 8.0)) * 8
    ch0 = max(8, min(per_tile - 8, ch0))
    return ch0, per_tile - ch0


def _stages(ch, cap):
    out = []
    off = 0
    while off < ch:
        n = min(cap, ch - off)
        out.append((off, n))
        off += n
    return out


@functools.partial(jax.jit, static_argnames=("n_pad", "n_out", "ch0", "ch1",
                                             "real_chunks", "with_cnt"))
def _sc_aggregate(src2d, dst2d, x, *, n_pad, n_out, ch0, ch1, real_chunks,
                  with_cnt):
    n_nodes, d = x.shape
    assert ch0 % 8 == 0 and ch1 % 8 == 0
    idx_cap = 64
    rpt = n_pad // _NS
    assert rpt % 8 == 0
    nz = (rpt + _CHUNK - 1) // _CHUNK

    mesh = plsc.VectorSubcoreMesh(core_axis_name="c", subcore_axis_name="s")

    out_type = [jax.ShapeDtypeStruct((_NC, n_out, d), jnp.float32)]
    if with_cnt:
        out_type.append(jax.ShapeDtypeStruct((_NC, n_out), jnp.float32))

    @functools.partial(
        pl.kernel,
        out_type=tuple(out_type),
        mesh=mesh,
        scratch_types=[
            pltpu.VMEM((idx_cap, _CHUNK), jnp.int32),
            pltpu.VMEM((idx_cap, _CHUNK), jnp.int32),
            pltpu.VMEM((_CHUNK, d), jnp.float32),
            pltpu.VMEM((_CHUNK, d), jnp.float32),
            pltpu.VMEM((_CHUNK,), jnp.float32),
            pltpu.VMEM_SHARED((n_pad, d), jnp.float32),
            pltpu.VMEM_SHARED((n_pad,), jnp.float32),
            pltpu.SemaphoreType.DMA,
            pltpu.SemaphoreType.DMA,
        ],
    )
    def agg_kernel(src_hbm, dst_hbm, x_hbm, agg_hbm, *rest):
        if with_cnt:
            cnt_hbm = rest[0]
            rest = rest[1:]
        sidx, didx, rows0, rows1, ones, acc, cacc, gsem, ssem = rest
        c = lax.axis_index("c")
        s = lax.axis_index("s")

        scope0 = jax.named_scope("init_zero")
        scope0.__enter__()

        @pl.loop(0, _CHUNK)
        def _(r):
            @pl.loop(0, d, step=_LANES)
            def _(col):
                rows0[r, pl.ds(col, _LANES)] = jnp.zeros((_LANES,),
                                                         jnp.float32)

        if with_cnt:
            @pl.loop(0, _CHUNK, step=_LANES)
            def _(i):
                ones[pl.ds(i, _LANES)] = jnp.ones((_LANES,), jnp.float32)

        @pl.loop(0, nz)
        def _(j):
            base = jnp.minimum(s * rpt + j * _CHUNK, n_pad - _CHUNK)
            pltpu.sync_copy(rows0, acc.at[pl.ds(base, _CHUNK)])
            if with_cnt:
                pltpu.sync_copy(rows0.at[0], cacc.at[pl.ds(base, _CHUNK)])

        plsc.subcore_barrier()
        scope0.__exit__(None, None, None)

        def fire_gather(t, buf):
            pltpu.async_copy(x_hbm.at[sidx.at[t]], buf, gsem)

        def drain_gather(t, buf):
            pltpu.make_async_copy(x_hbm.at[sidx.at[t]], buf, gsem).wait()

        def scatter_group(t, buf):
            descs = [pltpu.async_copy(buf, acc.at[didx.at[t]], ssem,
                                      add=True)]
            if with_cnt:
                descs.append(pltpu.async_copy(ones, cacc.at[didx.at[t]],
                                              ssem, add=True))
            for dsc in descs:
                dsc.wait()

        def run_stage(chunk_base, n_chunks, limit):
            pltpu.sync_copy(src_hbm.at[pl.ds(chunk_base, n_chunks)],
                            sidx.at[pl.ds(0, n_chunks)])
            pltpu.sync_copy(dst_hbm.at[pl.ds(chunk_base, n_chunks)],
                            didx.at[pl.ds(0, n_chunks)])

            @pl.when(limit > 0)
            def _():
                fire_gather(0, rows0)

            @pl.loop(0, n_chunks, step=2)
            def _(p):
                @pl.when(p < limit)
                def _():
                    drain_gather(p, rows0)

                @pl.when(p + 1 < limit)
                def _():
                    fire_gather(p + 1, rows1)

                @pl.when(p < limit)
                def _():
                    scatter_group(p, rows0)

                @pl.when(p + 1 < limit)
                def _():
                    drain_gather(p + 1, rows1)

                @pl.when(jnp.logical_and(p + 2 < n_chunks, p + 2 < limit))
                def _():
                    fire_gather(p + 2, rows0)

                @pl.when(p + 1 < limit)
                def _():
                    scatter_group(p + 1, rows1)

        with jax.named_scope("edge_loop"):
            @pl.when(c == 0)
            def _():
                for off, cnt_c in _stages(ch0, idx_cap):
                    base = s * ch0 + off
                    run_stage(base, cnt_c, real_chunks - base)

            @pl.when(c == 1)
            def _():
                for off, cnt_c in _stages(ch1, idx_cap):
                    base = _NS * ch0 + s * ch1 + off
                    run_stage(base, cnt_c, real_chunks - base)

        plsc.subcore_barrier()

        with jax.named_scope("writeout"):
            pltpu.sync_copy(acc.at[pl.ds(s * rpt, rpt)],
                            agg_hbm.at[c, pl.ds(s * rpt, rpt)])
            if with_cnt:
                @pl.when(s == 0)
                def _():
                    pltpu.sync_copy(cacc, cnt_hbm.at[c, pl.ds(0, n_pad)])

    return agg_kernel(src2d, dst2d, x)


def _dense_right(x, w_r, b_l, b_r):
    n, d = x.shape

    def body(x_ref, wr_ref, bl_ref, br_ref, o_ref):
        o_ref[...] = (jnp.dot(x_ref[...], wr_ref[...],
                              preferred_element_type=jnp.float32)
                      + bl_ref[...] + br_ref[...])

    return pl.pallas_call(
        body,
        out_shape=jax.ShapeDtypeStruct((n, d), jnp.float32),
    )(x, w_r, b_l, b_r)


def _dense_layer1(agg, cnt, xr, w_l, gamma, beta):
    n, d = xr.shape

    def body(agg_ref, cnt_ref, xr_ref, wl_ref, g_ref, be_ref, h_ref):
        cntv = jnp.maximum(cnt_ref[0, :n] + cnt_ref[1, :n], 1.0)
        mean = (agg_ref[0, :n, :] + agg_ref[1, :n, :]) / cntv[:, None]
        h = (jnp.dot(mean, wl_ref[...], preferred_element_type=jnp.float32)
             + xr_ref[...])
        mu = jnp.mean(h, axis=0, keepdims=True)
        var = jnp.mean((h - mu) * (h - mu), axis=0, keepdims=True)
        hn = (h - mu) * lax.rsqrt(var + 1e-5) * g_ref[...] + be_ref[...]
        h_ref[...] = jax.nn.gelu(hn)

    return pl.pallas_call(
        body,
        out_shape=jax.ShapeDtypeStruct((n, d), jnp.float32),
    )(agg, cnt, xr, w_l, gamma, beta)


def _dense_layer2(agg, cnt, hr, w_l):
    n, d = hr.shape

    def body(agg_ref, cnt_ref, hr_ref, wl_ref, o_ref):
        cntv = jnp.maximum(cnt_ref[0, :n] + cnt_ref[1, :n], 1.0)
        mean = (agg_ref[0, :n, :] + agg_ref[1, :n, :]) / cntv[:, None]
        o_ref[...] = (jnp.dot(mean, wl_ref[...],
                              preferred_element_type=jnp.float32)
                      + hr_ref[...])

    return pl.pallas_call(
        body,
        out_shape=jax.ShapeDtypeStruct((n, d), jnp.float32),
    )(agg, cnt, hr, w_l)


def kernel(x, edge_index, W_l1, b_l1, W_r1, b_r1, gamma1, beta1,
           W_l2, b_l2, W_r2, b_r2):
    n, d = x.shape
    e = edge_index.shape[1]

    n_pad = ((n + 16 + _CHUNK - 1) // _CHUNK) * _CHUNK
    n_out = ((n_pad + _NS * _CHUNK - 1) // (_NS * _CHUNK)) * (_NS * _CHUNK)
    nw = _NC * _NS
    unit = nw * _CHUNK * 8
    e_pad = ((e + unit - 1) // unit) * unit
    ch0, ch1 = _split_chunks(e_pad // _CHUNK)

    src = edge_index[0]
    dst = edge_index[1]
    if e_pad != e:
        pad = e_pad - e
        src = jnp.concatenate([src, jnp.zeros((pad,), jnp.int32)])
        trash = n + jnp.arange(pad, dtype=jnp.int32) % (n_pad - n)
        dst = jnp.concatenate([dst, trash])
    src2d = src.reshape(e_pad // _CHUNK, _CHUNK)
    dst2d = dst.reshape(e_pad // _CHUNK, _CHUNK)

    real_chunks = (e + _CHUNK - 1) // _CHUNK
    agg1, cnt = _sc_aggregate(src2d, dst2d, x, n_pad=n_pad, n_out=n_out,
                              ch0=ch0, ch1=ch1, real_chunks=real_chunks,
                              with_cnt=True)
    xr = _dense_right(x, W_r1, b_l1, b_r1)
    h = _dense_layer1(agg1, cnt, xr, W_l1, gamma1, beta1)
    (agg2,) = _sc_aggregate(src2d, dst2d, h, n_pad=n_pad, n_out=n_out,
                            ch0=ch0, ch1=ch1, real_chunks=real_chunks,
                            with_cnt=False)
    hr = _dense_right(h, W_r2, b_l2, b_r2)
    return _dense_layer2(agg2, cnt, hr, W_l2)
